# Initial kernel scaffold; baseline (speedup 1.0000x reference)
import jax
import jax.numpy as jnp
from jax.experimental import pallas as pl


def kernel(neighbor):
    raise NotImplementedError("write your pallas kernel here")


# speedup vs baseline: 1.1348x; 1.1348x over previous
import jax
import jax.numpy as jnp
from jax.experimental import pallas as pl


_ROWS_PER_BLOCK = 400


def _mean_body(x_ref, o_ref):
    o_ref[...] = jnp.mean(x_ref[...], axis=1)


def kernel(neighbor):
    n, k, d = neighbor.shape
    grid = (n // _ROWS_PER_BLOCK,)
    return pl.pallas_call(
        _mean_body,
        grid=grid,
        in_specs=[pl.BlockSpec((_ROWS_PER_BLOCK, k, d), lambda i: (i, 0, 0))],
        out_specs=pl.BlockSpec((_ROWS_PER_BLOCK, d), lambda i: (i, 0)),
        out_shape=jax.ShapeDtypeStruct((n, d), neighbor.dtype),
    )(neighbor)
